# baseline (speedup 1.0000x reference)
import functools

import jax
import jax.numpy as jnp
from jax import lax
from jax.experimental import pallas as pl
from jax.experimental.pallas import tpu as pltpu
from jax.experimental.pallas import tpu_sc as plsc

_NC = 2
_NS = 16
_L = 16
_NW = _NC * _NS


def _argmin_body(mu_ref, cpt_ref, out_ref, score_scr, bmin_ref, barg_ref, *,
                 tn, n_tiles, cb):
    i = pl.program_id(0)
    p = cpt_ref.shape[1]
    mu = mu_ref[...]
    mu_sq = jnp.sum(mu * mu, axis=1, keepdims=True)
    packed = jnp.concatenate([-2.0 * mu, mu_sq], axis=1)
    score_scr[...] = jnp.dot(
        packed, cpt_ref[...], preferred_element_type=jnp.float32
    )

    n_chunks = tn // 8
    inf = jnp.float32(jnp.inf)
    for c0 in range(0, p, cb):
        csl = pl.ds(c0, cb)

        def step(k, carry):
            rmin, rchk = carry
            blk = score_scr[pl.ds(k * 8, 8), csl]
            kf = jnp.broadcast_to(k.astype(jnp.float32), (8, cb))
            m = blk < rmin
            return jnp.where(m, blk, rmin), jnp.where(m, kf, rchk)

        rmin0 = score_scr[pl.ds(0, 8), csl]
        rchk0 = jnp.zeros((8, cb), jnp.float32)
        rmin, rchk = lax.fori_loop(
            1, n_chunks, step, (rmin0, rchk0), unroll=9
        )

        subl = lax.broadcasted_iota(jnp.int32, (8, cb), 0).astype(jnp.float32)
        row = rchk * 8.0 + subl

        for h in (4, 2, 1):
            va, ra = rmin[:h], row[:h]
            vb, rb = rmin[h:2 * h], row[h:2 * h]
            take = (vb < va) | ((vb == va) & (rb < ra))
            rmin = jnp.where(take, vb, va)
            row = jnp.where(take, rb, ra)
        lmin = rmin
        larg = row + jnp.float32(i * tn)

        @pl.when(i == 0)
        def _():
            bmin_ref[0:1, csl] = lmin
            barg_ref[0:1, csl] = larg

        @pl.when(i > 0)
        def _():
            bm = bmin_ref[0:1, csl]
            better = lmin < bm
            bmin_ref[0:1, csl] = jnp.where(better, lmin, bm)
            barg_ref[0:1, csl] = jnp.where(
                better, larg, barg_ref[0:1, csl]
            )

    @pl.when(i == n_tiles - 1)
    def _():
        out_ref[...] = barg_ref[...].astype(jnp.int32)


def _nearest_idx(positions, cp4t, tn):
    n, p = positions.shape[0], cp4t.shape[1]
    n_tiles = n // tn
    nn = pl.pallas_call(
        functools.partial(_argmin_body, tn=tn, n_tiles=n_tiles, cb=1024),
        grid=(n_tiles,),
        in_specs=[
            pl.BlockSpec((tn, 3), lambda i: (i, 0)),
            pl.BlockSpec((4, p), lambda i: (0, 0)),
        ],
        out_specs=pl.BlockSpec((1, p), lambda i: (0, 0)),
        out_shape=jax.ShapeDtypeStruct((1, p), jnp.int32),
        scratch_shapes=[
            pltpu.VMEM((tn, p), jnp.float32),
            pltpu.VMEM((1, p), jnp.float32),
            pltpu.VMEM((1, p), jnp.float32),
        ],
    )(positions, cp4t)
    return nn.reshape(p)


def _sc_residual_body(posf_hbm, scf_hbm, idx_hbm, cpx_hbm, cpy_hbm, cpz_hbm,
                      w_hbm, out_hbm,
                      idx_v, i3a_v, i3b_v, i3c_v, gmux_v, gmuy_v, gmuz_v,
                      gscx_v, gscy_v, gscz_v, cpx_v, cpy_v, cpz_v, w_v, acc_v,
                      sem, *, ppw):
    wid = lax.axis_index("s") * _NC + lax.axis_index("c")
    base = wid * ppw
    pltpu.sync_copy(idx_hbm.at[pl.ds(base, ppw)], idx_v)
    pltpu.sync_copy(cpx_hbm.at[pl.ds(base, ppw)], cpx_v)
    pltpu.sync_copy(cpy_hbm.at[pl.ds(base, ppw)], cpy_v)
    pltpu.sync_copy(cpz_hbm.at[pl.ds(base, ppw)], cpz_v)
    pltpu.sync_copy(w_hbm.at[pl.ds(base, ppw)], w_v)
    for g in range(ppw // _L):
        sl = pl.ds(g * _L, _L)
        i3 = idx_v[sl] * 3
        i3a_v[sl] = i3
        i3b_v[sl] = i3 + 1
        i3c_v[sl] = i3 + 2
    copies = [
        pltpu.async_copy(posf_hbm.at[i3a_v], gmux_v, sem),
        pltpu.async_copy(posf_hbm.at[i3b_v], gmuy_v, sem),
        pltpu.async_copy(posf_hbm.at[i3c_v], gmuz_v, sem),
        pltpu.async_copy(scf_hbm.at[i3a_v], gscx_v, sem),
        pltpu.async_copy(scf_hbm.at[i3b_v], gscy_v, sem),
        pltpu.async_copy(scf_hbm.at[i3c_v], gscz_v, sem),
    ]
    for c in copies:
        c.wait()

    acc = jnp.zeros((_L,), jnp.float32)
    for g in range(ppw // _L):
        sl = pl.ds(g * _L, _L)
        m2 = jnp.zeros((_L,), jnp.float32)
        for cp_v, gmu_v, gsc_v in (
            (cpx_v, gmux_v, gscx_v),
            (cpy_v, gmuy_v, gscy_v),
            (cpz_v, gmuz_v, gscz_v),
        ):
            delta = (cp_v[sl] - gmu_v[sl]) / (jnp.exp(gsc_v[sl]) + 1e-6)
            m2 = m2 + delta * delta
        seed = (lax.bitcast_convert_type(m2, jnp.int32) >> 1) + jnp.int32(
            0x1FBD1DF5
        )
        y = lax.bitcast_convert_type(seed, jnp.float32)
        for _ in range(3):
            y = 0.5 * (y + m2 / y)
        r = y - 1.0
        wv = jnp.clip(w_v[sl], 0.0, 1.0)
        acc = acc + r * r * wv
    acc_v[...] = acc
    pltpu.sync_copy(acc_v, out_hbm.at[wid])


def _sc_residual(positions, scales, nn_idx, cp4t, contact_confidence):
    n = positions.shape[0]
    p = cp4t.shape[1]
    ppw = p // _NW
    mesh = plsc.VectorSubcoreMesh(core_axis_name="c", subcore_axis_name="s")
    f32 = jnp.float32
    run = pl.kernel(
        functools.partial(_sc_residual_body, ppw=ppw),
        out_type=jax.ShapeDtypeStruct((_NW, _L), f32),
        mesh=mesh,
        scratch_types=[
            pltpu.VMEM((ppw,), jnp.int32),
            pltpu.VMEM((ppw,), jnp.int32),
            pltpu.VMEM((ppw,), jnp.int32),
            pltpu.VMEM((ppw,), jnp.int32),
            pltpu.VMEM((ppw,), f32),
            pltpu.VMEM((ppw,), f32),
            pltpu.VMEM((ppw,), f32),
            pltpu.VMEM((ppw,), f32),
            pltpu.VMEM((ppw,), f32),
            pltpu.VMEM((ppw,), f32),
            pltpu.VMEM((ppw,), f32),
            pltpu.VMEM((ppw,), f32),
            pltpu.VMEM((ppw,), f32),
            pltpu.VMEM((ppw,), f32),
            pltpu.VMEM((_L,), f32),
            pltpu.SemaphoreType.DMA,
        ],
    )
    return run(
        positions.reshape(3 * n),
        scales.reshape(3 * n),
        nn_idx,
        cp4t[0],
        cp4t[1],
        cp4t[2],
        contact_confidence,
    )


def kernel(positions, scales, contact_points, contact_normals,
           contact_confidence):
    del contact_normals
    p = contact_points.shape[0]
    cp4t = jnp.concatenate(
        [contact_points, jnp.ones((p, 1), jnp.float32)], axis=1
    ).T
    nn_idx = _nearest_idx(positions, cp4t, tn=512)
    partials = _sc_residual(positions, scales, nn_idx, cp4t,
                            contact_confidence)
    return jnp.sum(partials) / jnp.float32(p)


# speedup vs baseline: 1.2864x; 1.2864x over previous
import functools

import jax
import jax.numpy as jnp
from jax import lax
from jax.experimental import pallas as pl
from jax.experimental.pallas import tpu as pltpu
from jax.experimental.pallas import tpu_sc as plsc

_NC = 2
_NS = 16
_L = 16
_NW = _NC * _NS


def _argmin_body(mu_ref, cpt_ref, out_ref, bmin_ref, barg_ref, *, tn,
                 n_tiles):
    i = pl.program_id(0)
    mu = mu_ref[...]
    mu_sq = jnp.sum(mu * mu, axis=1, keepdims=True)
    ones = jnp.ones((mu.shape[0], 1), jnp.float32)
    packed = jnp.concatenate([-2.0 * mu, mu_sq, ones], axis=1)
    score = jnp.dot(
        packed, cpt_ref[...], preferred_element_type=jnp.float32
    )
    lmin = jnp.min(score, axis=0, keepdims=True)
    rowsf = lax.broadcasted_iota(jnp.int32, score.shape, 0).astype(
        jnp.float32
    )
    inf = jnp.float32(jnp.inf)
    larg = jnp.min(
        jnp.where(score == lmin, rowsf, inf), axis=0, keepdims=True
    ) + jnp.float32(i * tn)

    @pl.when(i == 0)
    def _():
        bmin_ref[...] = lmin
        barg_ref[...] = larg

    @pl.when(i > 0)
    def _():
        better = lmin < bmin_ref[...]
        bmin_ref[...] = jnp.where(better, lmin, bmin_ref[...])
        barg_ref[...] = jnp.where(better, larg, barg_ref[...])

    @pl.when(i == n_tiles - 1)
    def _():
        out_ref[...] = barg_ref[...].astype(jnp.int32)


def _nearest_idx(positions, cpt5, tn):
    n, p = positions.shape[0], cpt5.shape[1]
    n_tiles = n // tn
    nn = pl.pallas_call(
        functools.partial(_argmin_body, tn=tn, n_tiles=n_tiles),
        grid=(n_tiles,),
        in_specs=[
            pl.BlockSpec((tn, 3), lambda i: (i, 0)),
            pl.BlockSpec((5, p), lambda i: (0, 0)),
        ],
        out_specs=pl.BlockSpec((1, p), lambda i: (0, 0)),
        out_shape=jax.ShapeDtypeStruct((1, p), jnp.int32),
        scratch_shapes=[
            pltpu.VMEM((1, p), jnp.float32),
            pltpu.VMEM((1, p), jnp.float32),
        ],
    )(positions, cpt5)
    return nn.reshape(p)


def _sc_residual_body(mux_hbm, muy_hbm, muz_hbm, scx_hbm, scy_hbm, scz_hbm,
                      idx_hbm, cpx_hbm, cpy_hbm, cpz_hbm, w_hbm, out_hbm,
                      idx_v, gmux_v, gmuy_v, gmuz_v, gscx_v, gscy_v, gscz_v,
                      cpx_v, cpy_v, cpz_v, w_v, acc_v, sem, *, ppw):
    wid = lax.axis_index("s") * _NC + lax.axis_index("c")
    base = wid * ppw
    pltpu.sync_copy(idx_hbm.at[pl.ds(base, ppw)], idx_v)
    pltpu.sync_copy(cpx_hbm.at[pl.ds(base, ppw)], cpx_v)
    pltpu.sync_copy(cpy_hbm.at[pl.ds(base, ppw)], cpy_v)
    pltpu.sync_copy(cpz_hbm.at[pl.ds(base, ppw)], cpz_v)
    pltpu.sync_copy(w_hbm.at[pl.ds(base, ppw)], w_v)
    copies = [
        pltpu.async_copy(mux_hbm.at[idx_v], gmux_v, sem),
        pltpu.async_copy(muy_hbm.at[idx_v], gmuy_v, sem),
        pltpu.async_copy(muz_hbm.at[idx_v], gmuz_v, sem),
        pltpu.async_copy(scx_hbm.at[idx_v], gscx_v, sem),
        pltpu.async_copy(scy_hbm.at[idx_v], gscy_v, sem),
        pltpu.async_copy(scz_hbm.at[idx_v], gscz_v, sem),
    ]
    for c in copies:
        c.wait()

    acc = jnp.zeros((_L,), jnp.float32)
    for g in range(ppw // _L):
        sl = pl.ds(g * _L, _L)
        m2 = jnp.zeros((_L,), jnp.float32)
        for cp_v, gmu_v, gsc_v in (
            (cpx_v, gmux_v, gscx_v),
            (cpy_v, gmuy_v, gscy_v),
            (cpz_v, gmuz_v, gscz_v),
        ):
            delta = (cp_v[sl] - gmu_v[sl]) / (jnp.exp(gsc_v[sl]) + 1e-6)
            m2 = m2 + delta * delta
        seed = (lax.bitcast_convert_type(m2, jnp.int32) >> 1) + jnp.int32(
            0x1FBD1DF5
        )
        y = lax.bitcast_convert_type(seed, jnp.float32)
        for _ in range(3):
            y = 0.5 * (y + m2 / y)
        r = y - 1.0
        wv = jnp.clip(w_v[sl], 0.0, 1.0)
        acc = acc + r * r * wv
    acc_v[...] = acc
    pltpu.sync_copy(acc_v, out_hbm.at[wid])


def _sc_residual(mu_comps, sc_comps, nn_idx, cp_comps, contact_confidence,
                 p):
    ppw = p // _NW
    mesh = plsc.VectorSubcoreMesh(core_axis_name="c", subcore_axis_name="s")
    f32 = jnp.float32
    run = pl.kernel(
        functools.partial(_sc_residual_body, ppw=ppw),
        out_type=jax.ShapeDtypeStruct((_NW, _L), f32),
        mesh=mesh,
        scratch_types=[
            pltpu.VMEM((ppw,), jnp.int32),
            pltpu.VMEM((ppw,), f32),
            pltpu.VMEM((ppw,), f32),
            pltpu.VMEM((ppw,), f32),
            pltpu.VMEM((ppw,), f32),
            pltpu.VMEM((ppw,), f32),
            pltpu.VMEM((ppw,), f32),
            pltpu.VMEM((ppw,), f32),
            pltpu.VMEM((ppw,), f32),
            pltpu.VMEM((ppw,), f32),
            pltpu.VMEM((ppw,), f32),
            pltpu.VMEM((_L,), f32),
            pltpu.SemaphoreType.DMA,
        ],
    )
    return run(*mu_comps, *sc_comps, nn_idx, *cp_comps, contact_confidence)


def kernel(positions, scales, contact_points, contact_normals,
           contact_confidence):
    del contact_normals
    p = contact_points.shape[0]
    cp_sq = jnp.sum(contact_points * contact_points, axis=1, keepdims=True)
    cpt5 = jnp.concatenate(
        [contact_points, jnp.ones((p, 1), jnp.float32), cp_sq], axis=1
    ).T
    nn_idx = _nearest_idx(positions, cpt5, tn=512)
    mu_comps = (positions[:, 0], positions[:, 1], positions[:, 2])
    sc_comps = (scales[:, 0], scales[:, 1], scales[:, 2])
    cp_comps = (
        contact_points[:, 0], contact_points[:, 1], contact_points[:, 2],
    )
    partials = _sc_residual(mu_comps, sc_comps, nn_idx, cp_comps,
                            contact_confidence, p)
    return jnp.sum(partials) / jnp.float32(p)
